# baseline (speedup 1.0000x reference)
import functools

import jax
import jax.numpy as jnp
from jax import lax
from jax.experimental import pallas as pl
from jax.experimental.pallas import tpu as pltpu
from jax.experimental.pallas import tpu_sc as plsc

D = 768
H = 12
DH = D // H
E = 8
K = 3
S = 2048
FCH = 4
FBLK = 2 * D // FCH


def _mmT(a, w):
    return jax.lax.dot_general(a, w, (((1,), (1,)), ((), ())),
                               preferred_element_type=jnp.float32)


def _stage1(text_ref, ctx_ref, mod0_ref,
            rW1_ref, rb1_ref, rW2_ref, rb2_ref, rW3_ref, rb3_ref,
            in_proj_w_ref, in_proj_b_ref,
            out_w_ref, out_b_ref,
            gW1_ref, gb1_ref, gW2_ref, gb2_ref,
            f_ref, routing_ref, g16_ref):
    text = text_ref[...]
    mod0 = mod0_ref[...]
    enh = text + mod0

    summary = jnp.mean(enh, axis=0, keepdims=True)
    h1 = (_mmT(summary, rW1_ref[:, 0:D])
          + _mmT(ctx_ref[...], rW1_ref[:, D:2 * D]))
    h1 = jnp.maximum(h1 + rb1_ref[...], 0.0)
    h2 = jnp.maximum(_mmT(h1, rW2_ref[...]) + rb2_ref[...], 0.0)
    prod = h2 * rW3_ref[...]
    ones_mat = jnp.ones((D // 2, 128), jnp.float32)
    lg = jax.lax.dot_general(prod, ones_mat, (((1,), (0,)), ((), ())),
                             preferred_element_type=jnp.float32)
    rw = jax.nn.sigmoid(lg + rb3_ref[...])
    routing_ref[...] = rw / rw

    q0 = _mmT(enh[0:1], in_proj_w_ref[0:D, :]) + in_proj_b_ref[:, 0:D]
    h_iota = jax.lax.broadcasted_iota(jnp.int32, (H, D), 0)
    j_iota = jax.lax.broadcasted_iota(jnp.int32, (H, D), 1)
    head_mask = (j_iota // DH) == h_iota
    q_rows = jnp.where(head_mask, q0, 0.0)
    U = jax.lax.dot_general(q_rows, in_proj_w_ref[D:2 * D, :],
                            (((1,), (0,)), ((), ())),
                            preferred_element_type=jnp.float32)
    scores = _mmT(enh, U) * (1.0 / jnp.sqrt(jnp.float32(DH)))
    probs = jax.nn.softmax(scores, axis=0)
    P = jax.lax.dot_general(probs, enh, (((0,), (0,)), ((), ())),
                            preferred_element_type=jnp.float32)
    O = _mmT(P, in_proj_w_ref[2 * D:3 * D, :])
    o0 = jnp.sum(jnp.where(head_mask, O, 0.0), axis=0, keepdims=True)
    o0 = o0 + in_proj_b_ref[:, 2 * D:3 * D]
    f = _mmT(o0, out_w_ref[...]) + out_b_ref[...]
    f_ref[...] = f

    g1 = jnp.maximum(_mmT(f, gW1_ref[...]) + gb1_ref[...], 0.0)
    g16_ref[...] = _mmT(g1, gW2_ref[...]) + gb2_ref[...]


def _sc_gate(g_hbm, wsel_hbm, isel_hbm, g_v, w_v, i_v, e_v):
    tile0 = (lax.axis_index("c") == 0) & (lax.axis_index("s") == 0)
    lane = lax.iota(jnp.int32, 16)

    @pl.when(tile0)
    def _():
        pltpu.sync_copy(g_hbm, g_v)

        g = g_v[...]
        gs = [g[j] for j in range(E)]

        m = gs[0]
        for j in range(1, E):
            m = jnp.maximum(m, gs[j])
        v1 = gs[0]
        i1 = jnp.int32(0)
        for j in range(1, E):
            b = gs[j] > v1
            v1 = jnp.where(b, gs[j], v1)
            i1 = jnp.where(b, jnp.int32(j), i1)
        v2 = jnp.float32(-jnp.inf)
        i2 = jnp.int32(0)
        for j in range(E):
            b = (gs[j] > v2) & (jnp.int32(j) != i1)
            v2 = jnp.where(b, gs[j], v2)
            i2 = jnp.where(b, jnp.int32(j), i2)
        v3 = jnp.float32(-jnp.inf)
        i3 = jnp.int32(0)
        for j in range(E):
            b = (gs[j] > v3) & (jnp.int32(j) != i1) & (jnp.int32(j) != i2)
            v3 = jnp.where(b, gs[j], v3)
            i3 = jnp.where(b, jnp.int32(j), i3)

        e_vec = jnp.exp(g - m)
        e_v[...] = e_vec
        ssum = e_vec[0]
        for j in range(1, E):
            ssum = ssum + e_vec[j]
        gp = e_vec / ssum

        sel = (lane == i1) | (lane == i2) | (lane == i3)
        e2 = jnp.exp(jnp.where(sel, gp, -jnp.inf))
        s3 = e2[0]
        for j in range(1, E):
            s3 = s3 + e2[j]

        w_v[...] = jnp.where(sel, e2 / s3, 0.0)
        i_v[...] = jnp.where(lane == 0, i1,
                             jnp.where(lane == 1, i2,
                                       jnp.where(lane == 2, i3, 0)))
        pltpu.sync_copy(w_v, wsel_hbm)
        pltpu.sync_copy(i_v, isel_hbm)


def _stage2(idx_ref, f_ref, wsel_ref,
            eW1_ref, eb1_ref, eW2_ref, eb2_ref, eG_ref, eB_ref,
            out_ref, acc_ref):
    e = pl.program_id(0)
    c = pl.program_id(1)
    f = f_ref[...]
    eh = _mmT(f, eW1_ref[0]) + eb1_ref[0]
    eh = 0.5 * eh * (1.0 + jax.lax.erf(eh * (1.0 / jnp.sqrt(jnp.float32(2.0)))))
    part = _mmT(eh, eW2_ref[0])

    @pl.when(c == 0)
    def _():
        acc_ref[...] = jnp.zeros_like(acc_ref)

    acc_ref[...] += part

    @pl.when(c == FCH - 1)
    def _():
        eo = acc_ref[...] + eb2_ref[0]
        mu = jnp.mean(eo, axis=1, keepdims=True)
        cc = eo - mu
        var = jnp.mean(cc * cc, axis=1, keepdims=True)
        ln = cc / jnp.sqrt(var + 1e-5) * eG_ref[0] + eB_ref[0]
        iota_p = jax.lax.broadcasted_iota(jnp.int32, (1, 16), 1)
        w = jnp.sum(jnp.where(iota_p == idx_ref[e], wsel_ref[...], 0.0))

        @pl.when(e == 0)
        def _():
            out_ref[...] = jnp.zeros_like(out_ref)

        out_ref[...] += w * ln


def kernel(text, context, mod_emb, rW1, rb1, rW2, rb2, rW3, rb3,
           in_proj_w, in_proj_b, out_w, out_b,
           gW1, gb1, gW2, gb2, eW1, eb1, eW2, eb2, eG, eB):
    ctx = context.reshape(1, D)
    mod0 = mod_emb[0:1, :]

    gW2_pad = jnp.zeros((16, D // 2), jnp.float32).at[0:E].set(gW2)
    gb2_pad = jnp.full((1, 16), -jnp.inf, jnp.float32).at[0, 0:E].set(gb2)

    f, routing, g16 = pl.pallas_call(
        _stage1,
        out_shape=(
            jax.ShapeDtypeStruct((1, D), jnp.float32),
            jax.ShapeDtypeStruct((1, 128), jnp.float32),
            jax.ShapeDtypeStruct((1, 16), jnp.float32),
        ),
    )(text, ctx, mod0,
      rW1, rb1.reshape(1, D),
      rW2, rb2.reshape(1, D // 2), rW3,
      jnp.broadcast_to(rb3.reshape(1, 1), (1, 128)),
      in_proj_w, in_proj_b.reshape(1, 3 * D),
      out_w, out_b.reshape(1, D),
      gW1, gb1.reshape(1, D // 2), gW2_pad, gb2_pad)

    sc_gate = functools.partial(
        pl.kernel,
        mesh=plsc.VectorSubcoreMesh(core_axis_name="c", subcore_axis_name="s"),
        out_type=(
            jax.ShapeDtypeStruct((16,), jnp.float32),
            jax.ShapeDtypeStruct((16,), jnp.int32),
        ),
        scratch_types=[
            pltpu.VMEM((16,), jnp.float32),
            pltpu.VMEM((16,), jnp.float32),
            pltpu.VMEM((16,), jnp.int32),
            pltpu.VMEM((16,), jnp.float32),
        ],
    )(_sc_gate)
    wsel16, isel16 = sc_gate(g16.reshape(16))

    wsel = wsel16.reshape(1, 16)
    top_i = isel16[0:K]

    fused = pl.pallas_call(
        _stage2,
        grid_spec=pltpu.PrefetchScalarGridSpec(
            num_scalar_prefetch=1,
            grid=(K, FCH),
            in_specs=[
                pl.BlockSpec((1, D), lambda e, c, idx: (0, 0)),
                pl.BlockSpec((1, 16), lambda e, c, idx: (0, 0)),
                pl.BlockSpec((1, FBLK, D), lambda e, c, idx: (idx[e], c, 0)),
                pl.BlockSpec((1, 1, FBLK), lambda e, c, idx: (idx[e], 0, c)),
                pl.BlockSpec((1, D, FBLK), lambda e, c, idx: (idx[e], 0, c)),
                pl.BlockSpec((1, 1, D), lambda e, c, idx: (idx[e], 0, 0)),
                pl.BlockSpec((1, 1, D), lambda e, c, idx: (idx[e], 0, 0)),
                pl.BlockSpec((1, 1, D), lambda e, c, idx: (idx[e], 0, 0)),
            ],
            out_specs=pl.BlockSpec((1, D), lambda e, c, idx: (0, 0)),
            scratch_shapes=[pltpu.VMEM((1, D), jnp.float32)],
        ),
        out_shape=jax.ShapeDtypeStruct((1, D), jnp.float32),
        compiler_params=pltpu.CompilerParams(
            dimension_semantics=("arbitrary", "arbitrary")),
    )(top_i, f, wsel,
      eW1, eb1.reshape(E, 1, 2 * D), eW2,
      eb2.reshape(E, 1, D), eG.reshape(E, 1, D), eB.reshape(E, 1, D))

    return fused.reshape(D), routing[0, 0]


# speedup vs baseline: 1.0070x; 1.0070x over previous
import functools

import jax
import jax.numpy as jnp
from jax import lax
from jax.experimental import pallas as pl
from jax.experimental.pallas import tpu as pltpu
from jax.experimental.pallas import tpu_sc as plsc

D = 768
H = 12
DH = D // H
E = 8
K = 3
S = 2048
FCH = 4
FBLK = 2 * D // FCH


def _mmT(a, w):
    return jax.lax.dot_general(a, w, (((1,), (1,)), ((), ())),
                               preferred_element_type=jnp.float32)


def _stage1(text_ref, mod0_ref,
            in_proj_w_ref, in_proj_b_ref,
            out_w_ref, out_b_ref,
            gW1_ref, gb1_ref, gW2_ref, gb2_ref,
            f_ref, summary_ref, g16_ref):
    text = text_ref[...]
    mod0 = mod0_ref[...]
    enh = text + mod0

    summary_ref[...] = jnp.mean(enh, axis=0, keepdims=True)

    q0 = _mmT(enh[0:1], in_proj_w_ref[0:D, :]) + in_proj_b_ref[:, 0:D]
    h_iota = jax.lax.broadcasted_iota(jnp.int32, (H, D), 0)
    j_iota = jax.lax.broadcasted_iota(jnp.int32, (H, D), 1)
    head_mask = (j_iota // DH) == h_iota
    q_rows = jnp.where(head_mask, q0, 0.0)
    U = jax.lax.dot_general(q_rows, in_proj_w_ref[D:2 * D, :],
                            (((1,), (0,)), ((), ())),
                            preferred_element_type=jnp.float32)
    scores = _mmT(enh, U) * (1.0 / jnp.sqrt(jnp.float32(DH)))
    probs = jax.nn.softmax(scores, axis=0)
    P = jax.lax.dot_general(probs, enh, (((0,), (0,)), ((), ())),
                            preferred_element_type=jnp.float32)
    O = _mmT(P, in_proj_w_ref[2 * D:3 * D, :])
    o0 = jnp.sum(jnp.where(head_mask, O, 0.0), axis=0, keepdims=True)
    o0 = o0 + in_proj_b_ref[:, 2 * D:3 * D]
    f = _mmT(o0, out_w_ref[...]) + out_b_ref[...]
    f_ref[...] = f

    g1 = jnp.maximum(_mmT(f, gW1_ref[...]) + gb1_ref[...], 0.0)
    g16_ref[...] = _mmT(g1, gW2_ref[...]) + gb2_ref[...]


def _router(summary_ref, ctx_ref,
            rW1_ref, rb1_ref, rW2_ref, rb2_ref, rW3_ref, rb3_ref,
            routing_ref):
    h1 = (_mmT(summary_ref[...], rW1_ref[:, 0:D])
          + _mmT(ctx_ref[...], rW1_ref[:, D:2 * D]))
    h1 = jnp.maximum(h1 + rb1_ref[...], 0.0)
    h2 = jnp.maximum(_mmT(h1, rW2_ref[...]) + rb2_ref[...], 0.0)
    prod = h2 * rW3_ref[...]
    ones_mat = jnp.ones((D // 2, 128), jnp.float32)
    lg = jax.lax.dot_general(prod, ones_mat, (((1,), (0,)), ((), ())),
                             preferred_element_type=jnp.float32)
    rw = jax.nn.sigmoid(lg + rb3_ref[...])
    routing_ref[...] = rw / rw


def _sc_gate(g_hbm, wsel_hbm, isel_hbm, g_v, w_v, i_v, e_v):
    tile0 = (lax.axis_index("c") == 0) & (lax.axis_index("s") == 0)
    lane = lax.iota(jnp.int32, 16)

    @pl.when(tile0)
    def _():
        pltpu.sync_copy(g_hbm, g_v)

        g = g_v[...]
        gs = [g[j] for j in range(E)]

        m = gs[0]
        for j in range(1, E):
            m = jnp.maximum(m, gs[j])
        v1 = gs[0]
        i1 = jnp.int32(0)
        for j in range(1, E):
            b = gs[j] > v1
            v1 = jnp.where(b, gs[j], v1)
            i1 = jnp.where(b, jnp.int32(j), i1)
        v2 = jnp.float32(-jnp.inf)
        i2 = jnp.int32(0)
        for j in range(E):
            b = (gs[j] > v2) & (jnp.int32(j) != i1)
            v2 = jnp.where(b, gs[j], v2)
            i2 = jnp.where(b, jnp.int32(j), i2)
        v3 = jnp.float32(-jnp.inf)
        i3 = jnp.int32(0)
        for j in range(E):
            b = (gs[j] > v3) & (jnp.int32(j) != i1) & (jnp.int32(j) != i2)
            v3 = jnp.where(b, gs[j], v3)
            i3 = jnp.where(b, jnp.int32(j), i3)

        e_vec = jnp.exp(g - m)
        e_v[...] = e_vec
        ssum = e_vec[0]
        for j in range(1, E):
            ssum = ssum + e_vec[j]
        gp = e_vec / ssum

        sel = (lane == i1) | (lane == i2) | (lane == i3)
        e2 = jnp.exp(jnp.where(sel, gp, -jnp.inf))
        s3 = e2[0]
        for j in range(1, E):
            s3 = s3 + e2[j]

        w_v[...] = jnp.where(sel, e2 / s3, 0.0)
        i_v[...] = jnp.where(lane == 0, i1,
                             jnp.where(lane == 1, i2,
                                       jnp.where(lane == 2, i3, 0)))
        pltpu.sync_copy(w_v, wsel_hbm)
        pltpu.sync_copy(i_v, isel_hbm)


def _stage2(idx_ref, f_ref, wsel_ref,
            eW1_ref, eb1_ref, eW2_ref, eb2_ref, eG_ref, eB_ref,
            out_ref, acc_ref):
    e = pl.program_id(0)
    c = pl.program_id(1)
    f = f_ref[...]
    eh = _mmT(f, eW1_ref[0]) + eb1_ref[0]
    eh = 0.5 * eh * (1.0 + jax.lax.erf(eh * (1.0 / jnp.sqrt(jnp.float32(2.0)))))
    part = _mmT(eh, eW2_ref[0])

    @pl.when(c == 0)
    def _():
        acc_ref[...] = jnp.zeros_like(acc_ref)

    acc_ref[...] += part

    @pl.when(c == FCH - 1)
    def _():
        eo = acc_ref[...] + eb2_ref[0]
        mu = jnp.mean(eo, axis=1, keepdims=True)
        cc = eo - mu
        var = jnp.mean(cc * cc, axis=1, keepdims=True)
        ln = cc / jnp.sqrt(var + 1e-5) * eG_ref[0] + eB_ref[0]
        iota_p = jax.lax.broadcasted_iota(jnp.int32, (1, 16), 1)
        w = jnp.sum(jnp.where(iota_p == idx_ref[e], wsel_ref[...], 0.0))

        @pl.when(e == 0)
        def _():
            out_ref[...] = jnp.zeros_like(out_ref)

        out_ref[...] += w * ln


def kernel(text, context, mod_emb, rW1, rb1, rW2, rb2, rW3, rb3,
           in_proj_w, in_proj_b, out_w, out_b,
           gW1, gb1, gW2, gb2, eW1, eb1, eW2, eb2, eG, eB):
    ctx = context.reshape(1, D)
    mod0 = mod_emb[0:1, :]

    gW2_pad = jnp.zeros((16, D // 2), jnp.float32).at[0:E].set(gW2)
    gb2_pad = jnp.full((1, 16), -jnp.inf, jnp.float32).at[0, 0:E].set(gb2)

    f, summary, g16 = pl.pallas_call(
        _stage1,
        out_shape=(
            jax.ShapeDtypeStruct((1, D), jnp.float32),
            jax.ShapeDtypeStruct((1, D), jnp.float32),
            jax.ShapeDtypeStruct((1, 16), jnp.float32),
        ),
    )(text, mod0,
      in_proj_w, in_proj_b.reshape(1, 3 * D),
      out_w, out_b.reshape(1, D),
      gW1, gb1.reshape(1, D // 2), gW2_pad, gb2_pad)

    routing = pl.pallas_call(
        _router,
        out_shape=jax.ShapeDtypeStruct((1, 128), jnp.float32),
    )(summary, ctx,
      rW1, rb1.reshape(1, D),
      rW2, rb2.reshape(1, D // 2), rW3,
      jnp.broadcast_to(rb3.reshape(1, 1), (1, 128)))

    sc_gate = functools.partial(
        pl.kernel,
        mesh=plsc.VectorSubcoreMesh(core_axis_name="c", subcore_axis_name="s"),
        out_type=(
            jax.ShapeDtypeStruct((16,), jnp.float32),
            jax.ShapeDtypeStruct((16,), jnp.int32),
        ),
        scratch_types=[
            pltpu.VMEM((16,), jnp.float32),
            pltpu.VMEM((16,), jnp.float32),
            pltpu.VMEM((16,), jnp.int32),
            pltpu.VMEM((16,), jnp.float32),
        ],
    )(_sc_gate)
    wsel16, isel16 = sc_gate(g16.reshape(16))

    wsel = wsel16.reshape(1, 16)
    top_i = isel16[0:K]

    fused = pl.pallas_call(
        _stage2,
        grid_spec=pltpu.PrefetchScalarGridSpec(
            num_scalar_prefetch=1,
            grid=(K, FCH),
            in_specs=[
                pl.BlockSpec((1, D), lambda e, c, idx: (0, 0)),
                pl.BlockSpec((1, 16), lambda e, c, idx: (0, 0)),
                pl.BlockSpec((1, FBLK, D), lambda e, c, idx: (idx[e], c, 0)),
                pl.BlockSpec((1, 1, FBLK), lambda e, c, idx: (idx[e], 0, c)),
                pl.BlockSpec((1, D, FBLK), lambda e, c, idx: (idx[e], 0, c)),
                pl.BlockSpec((1, 1, D), lambda e, c, idx: (idx[e], 0, 0)),
                pl.BlockSpec((1, 1, D), lambda e, c, idx: (idx[e], 0, 0)),
                pl.BlockSpec((1, 1, D), lambda e, c, idx: (idx[e], 0, 0)),
            ],
            out_specs=pl.BlockSpec((1, D), lambda e, c, idx: (0, 0)),
            scratch_shapes=[pltpu.VMEM((1, D), jnp.float32)],
        ),
        out_shape=jax.ShapeDtypeStruct((1, D), jnp.float32),
        compiler_params=pltpu.CompilerParams(
            dimension_semantics=("arbitrary", "arbitrary")),
    )(top_i, f, wsel,
      eW1, eb1.reshape(E, 1, 2 * D), eW2,
      eb2.reshape(E, 1, D), eG.reshape(E, 1, D), eB.reshape(E, 1, D))

    return fused.reshape(D), routing[0, 0]


# speedup vs baseline: 1.0142x; 1.0072x over previous
import functools

import jax
import jax.numpy as jnp
from jax import lax
from jax.experimental import pallas as pl
from jax.experimental.pallas import tpu as pltpu
from jax.experimental.pallas import tpu_sc as plsc

D = 768
H = 12
DH = D // H
E = 8
K = 3
S = 2048
FCH = 4
FBLK = 2 * D // FCH


def _mmT(a, w):
    return jax.lax.dot_general(a, w, (((1,), (1,)), ((), ())),
                               preferred_element_type=jnp.float32)


def _stage1(text_ref, mod0_ref,
            in_proj_w_ref, in_proj_b_ref,
            out_w_ref, out_b_ref,
            gW1_ref, gb1_ref, gW2_ref, gb2_ref,
            f_ref, summary_ref, g16_ref):
    text = text_ref[...]
    mod0 = mod0_ref[...]
    enh = text + mod0

    summary_ref[...] = jnp.mean(enh, axis=0, keepdims=True)

    q0 = _mmT(enh[0:1], in_proj_w_ref[0:D, :]) + in_proj_b_ref[:, 0:D]
    h_iota = jax.lax.broadcasted_iota(jnp.int32, (H, D), 0)
    j_iota = jax.lax.broadcasted_iota(jnp.int32, (H, D), 1)
    head_mask = (j_iota // DH) == h_iota
    q_rows = jnp.where(head_mask, q0, 0.0)
    U = jax.lax.dot_general(q_rows, in_proj_w_ref[D:2 * D, :],
                            (((1,), (0,)), ((), ())),
                            preferred_element_type=jnp.float32)
    scores = _mmT(enh, U) * (1.0 / jnp.sqrt(jnp.float32(DH)))
    probs = jax.nn.softmax(scores, axis=0)
    P = jax.lax.dot_general(probs, enh, (((0,), (0,)), ((), ())),
                            preferred_element_type=jnp.float32)
    O = _mmT(P, in_proj_w_ref[2 * D:3 * D, :])
    o0 = jnp.sum(jnp.where(head_mask, O, 0.0), axis=0, keepdims=True)
    o0 = o0 + in_proj_b_ref[:, 2 * D:3 * D]
    f = _mmT(o0, out_w_ref[...]) + out_b_ref[...]
    f_ref[...] = f

    g1 = jnp.maximum(_mmT(f, gW1_ref[...]) + gb1_ref[...], 0.0)
    g16_ref[...] = _mmT(g1, gW2_ref[...]) + gb2_ref[...]


def _sc_gate(g_hbm, wsel_hbm, isel_hbm, g_v, w_v, i_v, e_v):
    tile0 = (lax.axis_index("c") == 0) & (lax.axis_index("s") == 0)
    lane = lax.iota(jnp.int32, 16)

    @pl.when(tile0)
    def _():
        pltpu.sync_copy(g_hbm, g_v)

        g = g_v[...]
        gs = [g[j] for j in range(E)]

        m = gs[0]
        for j in range(1, E):
            m = jnp.maximum(m, gs[j])
        v1 = gs[0]
        i1 = jnp.int32(0)
        for j in range(1, E):
            b = gs[j] > v1
            v1 = jnp.where(b, gs[j], v1)
            i1 = jnp.where(b, jnp.int32(j), i1)
        v2 = jnp.float32(-jnp.inf)
        i2 = jnp.int32(0)
        for j in range(E):
            b = (gs[j] > v2) & (jnp.int32(j) != i1)
            v2 = jnp.where(b, gs[j], v2)
            i2 = jnp.where(b, jnp.int32(j), i2)
        v3 = jnp.float32(-jnp.inf)
        i3 = jnp.int32(0)
        for j in range(E):
            b = (gs[j] > v3) & (jnp.int32(j) != i1) & (jnp.int32(j) != i2)
            v3 = jnp.where(b, gs[j], v3)
            i3 = jnp.where(b, jnp.int32(j), i3)

        e_vec = jnp.exp(g - m)
        e_v[...] = e_vec
        ssum = e_vec[0]
        for j in range(1, E):
            ssum = ssum + e_vec[j]
        gp = e_vec / ssum

        sel = (lane == i1) | (lane == i2) | (lane == i3)
        e2 = jnp.exp(jnp.where(sel, gp, -jnp.inf))
        s3 = e2[0]
        for j in range(1, E):
            s3 = s3 + e2[j]

        w_v[...] = jnp.where(sel, e2 / s3, 0.0)
        i_v[...] = jnp.where(lane == 0, i1,
                             jnp.where(lane == 1, i2,
                                       jnp.where(lane == 2, i3, 0)))
        pltpu.sync_copy(w_v, wsel_hbm)
        pltpu.sync_copy(i_v, isel_hbm)


def _stage2(idx_ref, f_ref, wsel_ref,
            eW1_ref, eb1_ref, eW2_ref, eb2_ref, eG_ref, eB_ref,
            comb_ref, rW1_ref, rb1_ref, rW2_ref, rb2_ref, rW3_ref, rb3_ref,
            out_ref, routing_ref, acc_ref, h1_ref):
    e = pl.program_id(0)
    c = pl.program_id(1)
    gi = e * FCH + c

    h1p = _mmT(comb_ref[:, pl.ds(gi * 128, 128)], rW1_ref[...])

    @pl.when(gi == 0)
    def _():
        h1_ref[...] = jnp.zeros_like(h1_ref)

    h1_ref[...] += h1p

    @pl.when(gi == K * FCH - 1)
    def _():
        h1 = jnp.maximum(h1_ref[...] + rb1_ref[...], 0.0)
        h2 = jnp.maximum(_mmT(h1, rW2_ref[...]) + rb2_ref[...], 0.0)
        prod = h2 * rW3_ref[...]
        ones_mat = jnp.ones((D // 2, 128), jnp.float32)
        lg = jax.lax.dot_general(prod, ones_mat, (((1,), (0,)), ((), ())),
                                 preferred_element_type=jnp.float32)
        rw = jax.nn.sigmoid(lg + rb3_ref[...])
        routing_ref[...] = rw / rw

    f = f_ref[...]
    eh = _mmT(f, eW1_ref[0]) + eb1_ref[0]
    eh = 0.5 * eh * (1.0 + jax.lax.erf(eh * (1.0 / jnp.sqrt(jnp.float32(2.0)))))
    part = _mmT(eh, eW2_ref[0])

    @pl.when(c == 0)
    def _():
        acc_ref[...] = jnp.zeros_like(acc_ref)

    acc_ref[...] += part

    @pl.when(c == FCH - 1)
    def _():
        eo = acc_ref[...] + eb2_ref[0]
        mu = jnp.mean(eo, axis=1, keepdims=True)
        cc = eo - mu
        var = jnp.mean(cc * cc, axis=1, keepdims=True)
        ln = cc / jnp.sqrt(var + 1e-5) * eG_ref[0] + eB_ref[0]
        iota_p = jax.lax.broadcasted_iota(jnp.int32, (1, 16), 1)
        w = jnp.sum(jnp.where(iota_p == idx_ref[e], wsel_ref[...], 0.0))

        @pl.when(e == 0)
        def _():
            out_ref[...] = jnp.zeros_like(out_ref)

        out_ref[...] += w * ln


def kernel(text, context, mod_emb, rW1, rb1, rW2, rb2, rW3, rb3,
           in_proj_w, in_proj_b, out_w, out_b,
           gW1, gb1, gW2, gb2, eW1, eb1, eW2, eb2, eG, eB):
    ctx = context.reshape(1, D)
    mod0 = mod_emb[0:1, :]

    gW2_pad = jnp.zeros((16, D // 2), jnp.float32).at[0:E].set(gW2)
    gb2_pad = jnp.full((1, 16), -jnp.inf, jnp.float32).at[0, 0:E].set(gb2)

    f, summary, g16 = pl.pallas_call(
        _stage1,
        out_shape=(
            jax.ShapeDtypeStruct((1, D), jnp.float32),
            jax.ShapeDtypeStruct((1, D), jnp.float32),
            jax.ShapeDtypeStruct((1, 16), jnp.float32),
        ),
    )(text, mod0,
      in_proj_w, in_proj_b.reshape(1, 3 * D),
      out_w, out_b.reshape(1, D),
      gW1, gb1.reshape(1, D // 2), gW2_pad, gb2_pad)

    combined = jnp.concatenate([summary, ctx], axis=1)

    sc_gate = functools.partial(
        pl.kernel,
        mesh=plsc.VectorSubcoreMesh(core_axis_name="c", subcore_axis_name="s"),
        out_type=(
            jax.ShapeDtypeStruct((16,), jnp.float32),
            jax.ShapeDtypeStruct((16,), jnp.int32),
        ),
        scratch_types=[
            pltpu.VMEM((16,), jnp.float32),
            pltpu.VMEM((16,), jnp.float32),
            pltpu.VMEM((16,), jnp.int32),
            pltpu.VMEM((16,), jnp.float32),
        ],
    )(_sc_gate)
    wsel16, isel16 = sc_gate(g16.reshape(16))

    wsel = wsel16.reshape(1, 16)
    top_i = isel16[0:K]

    fused, routing = pl.pallas_call(
        _stage2,
        grid_spec=pltpu.PrefetchScalarGridSpec(
            num_scalar_prefetch=1,
            grid=(K, FCH),
            in_specs=[
                pl.BlockSpec((1, D), lambda e, c, idx: (0, 0)),
                pl.BlockSpec((1, 16), lambda e, c, idx: (0, 0)),
                pl.BlockSpec((1, FBLK, D), lambda e, c, idx: (idx[e], c, 0)),
                pl.BlockSpec((1, 1, FBLK), lambda e, c, idx: (idx[e], 0, c)),
                pl.BlockSpec((1, D, FBLK), lambda e, c, idx: (idx[e], 0, c)),
                pl.BlockSpec((1, 1, D), lambda e, c, idx: (idx[e], 0, 0)),
                pl.BlockSpec((1, 1, D), lambda e, c, idx: (idx[e], 0, 0)),
                pl.BlockSpec((1, 1, D), lambda e, c, idx: (idx[e], 0, 0)),
                pl.BlockSpec((1, 2 * D), lambda e, c, idx: (0, 0)),
                pl.BlockSpec((D, 128), lambda e, c, idx: (0, e * FCH + c)),
                pl.BlockSpec((1, D), lambda e, c, idx: (0, 0)),
                pl.BlockSpec((D // 2, D), lambda e, c, idx: (0, 0)),
                pl.BlockSpec((1, D // 2), lambda e, c, idx: (0, 0)),
                pl.BlockSpec((1, D // 2), lambda e, c, idx: (0, 0)),
                pl.BlockSpec((1, 128), lambda e, c, idx: (0, 0)),
            ],
            out_specs=(
                pl.BlockSpec((1, D), lambda e, c, idx: (0, 0)),
                pl.BlockSpec((1, 128), lambda e, c, idx: (0, 0)),
            ),
            scratch_shapes=[pltpu.VMEM((1, D), jnp.float32),
                            pltpu.VMEM((1, D), jnp.float32)],
        ),
        out_shape=(jax.ShapeDtypeStruct((1, D), jnp.float32),
                   jax.ShapeDtypeStruct((1, 128), jnp.float32)),
        compiler_params=pltpu.CompilerParams(
            dimension_semantics=("arbitrary", "arbitrary")),
    )(top_i, f, wsel,
      eW1, eb1.reshape(E, 1, 2 * D), eW2,
      eb2.reshape(E, 1, D), eG.reshape(E, 1, D), eB.reshape(E, 1, D),
      combined, rW1, rb1.reshape(1, D),
      rW2, rb2.reshape(1, D // 2), rW3,
      jnp.broadcast_to(rb3.reshape(1, 1), (1, 128)))

    return fused.reshape(D), routing[0, 0]
